# baseline (speedup 1.0000x reference)
import functools
import math

import jax
import jax.numpy as jnp
from jax import lax
from jax.experimental import pallas as pl
from jax.experimental.pallas import tpu as pltpu
from jax.experimental.pallas import tpu_sc as plsc

KK = 8192
DD = 256
BETA_ = 0.25
MT = 512
KC = 2048

NC = 2
NS = 16
NW = NC * NS
NTOK = 32768
B_PER_W = NTOK // NW
GCH = 128
NCH = B_PER_W // GCH


def _dist_argmin_body(x_ref, e_ref, idx_ref, dmin_ref, en_ref, eb_ref,
                      iota_ref):
    @pl.when(pl.program_id(0) == 0)
    def _():
        en_ref[...] = jnp.sum(e_ref[...] ** 2, axis=1)[None, :]
        eb_ref[...] = e_ref[...].astype(jnp.bfloat16)
        iota_ref[...] = lax.broadcasted_iota(
            jnp.int32, (1, KC), 1).astype(jnp.float32)

    x = x_ref[...]
    xn = jnp.sum(x ** 2, axis=1, keepdims=True)
    xb = (-2.0 * x).astype(jnp.bfloat16)
    iota_f = iota_ref[...]
    run_min = jnp.full((MT, 1), jnp.inf, jnp.float32)
    run_idx = jnp.zeros((MT, 1), jnp.float32)
    for c in range(KK // KC):
        dot = lax.dot_general(xb, eb_ref[pl.ds(c * KC, KC), :],
                              (((1,), (1,)), ((), ())),
                              preferred_element_type=jnp.float32)
        s = (xn + dot) + en_ref[0, pl.ds(c * KC, KC)][None, :]
        mv = jnp.min(s, axis=1, keepdims=True)
        li = jnp.min(jnp.where(s == mv, iota_f, float(KK)),
                     axis=1, keepdims=True)
        better = mv < run_min
        run_idx = jnp.where(better, li + float(c * KC), run_idx)
        run_min = jnp.where(better, mv, run_min)
    idx_ref[...] = run_idx.astype(jnp.int32)
    dmin_ref[...] = run_min


def _dist_argmin(flat, embedding):
    n = flat.shape[0]
    return pl.pallas_call(
        _dist_argmin_body,
        grid=(n // MT,),
        in_specs=[pl.BlockSpec((MT, DD), lambda i: (i, 0)),
                  pl.BlockSpec((KK, DD), lambda i: (0, 0))],
        out_specs=[pl.BlockSpec((MT, 1), lambda i: (i, 0)),
                   pl.BlockSpec((MT, 1), lambda i: (i, 0))],
        out_shape=[jax.ShapeDtypeStruct((n, 1), jnp.int32),
                   jax.ShapeDtypeStruct((n, 1), jnp.float32)],
        scratch_shapes=[pltpu.VMEM((1, KK), jnp.float32),
                        pltpu.VMEM((KK, DD), jnp.bfloat16),
                        pltpu.VMEM((1, KC), jnp.float32)],
    )(flat, embedding)


def _sc_gather_hist_body(e_hbm, idx_hbm, ones_hbm, zeros_hbm,
                         zq_hbm, hist_hbm,
                         idx_v, rows_v, ones_v, hist_sh, gsem):
    cid = lax.axis_index("c")
    sid = lax.axis_index("s")
    wid = sid * NC + cid

    pltpu.sync_copy(idx_hbm.at[pl.ds(wid * NCH, NCH)], idx_v)
    pltpu.sync_copy(ones_hbm, ones_v)

    @pl.when(sid == 0)
    def _():
        pltpu.sync_copy(zeros_hbm, hist_sh)
    plsc.subcore_barrier()

    base = wid * B_PER_W
    for j in range(NCH):
        pltpu.async_copy(e_hbm.at[idx_v.at[j]], rows_v, gsem).wait()
        pltpu.sync_copy(rows_v, zq_hbm.at[pl.ds(base + j * GCH, GCH)])
        pltpu.sync_copy(ones_v, hist_sh.at[idx_v.at[j]], add=True)

    plsc.subcore_barrier()
    @pl.when(sid == 0)
    def _():
        pltpu.sync_copy(hist_sh, hist_hbm.at[cid])


def _sc_gather_hist(embedding, indices):
    idx2 = indices.reshape(NTOK // GCH, GCH)
    ones = jnp.ones((GCH,), jnp.float32)
    zeros = jnp.zeros((KK,), jnp.float32)
    mesh = plsc.VectorSubcoreMesh(core_axis_name="c", subcore_axis_name="s")
    f = pl.kernel(
        _sc_gather_hist_body,
        mesh=mesh,
        out_type=[jax.ShapeDtypeStruct((NTOK, DD), jnp.float32),
                  jax.ShapeDtypeStruct((NC, KK), jnp.float32)],
        scratch_types=[pltpu.VMEM((NCH, GCH), jnp.int32),
                       pltpu.VMEM((GCH, DD), jnp.float32),
                       pltpu.VMEM((GCH,), jnp.float32),
                       pltpu.VMEM_SHARED((KK,), jnp.float32),
                       pltpu.SemaphoreType.DMA],
    )
    return f(embedding, idx2, ones, zeros)


def _scalars_body(d_ref, h_ref, loss_ref, perp_ref):
    loss = BETA_ * (jnp.sum(d_ref[...]) / (NTOK * DD))
    loss_ref[...] = jnp.reshape(loss, (1, 1))
    u = jnp.sum(h_ref[...], axis=0, keepdims=True)
    total = jnp.maximum(jnp.sum(u), 1e-12)
    probs = jnp.clip(u / total, 1e-12, None)
    perp = jnp.exp(-jnp.sum(probs * jnp.log(probs)))
    perp_ref[...] = jnp.reshape(perp, (1, 1))


def _scalars(dmin2, hist):
    return pl.pallas_call(
        _scalars_body,
        out_shape=[jax.ShapeDtypeStruct((1, 1), jnp.float32),
                   jax.ShapeDtypeStruct((1, 1), jnp.float32)],
    )(dmin2.reshape(NTOK // KK, KK), hist)


def kernel(z_e, embedding):
    B, M, Dd = z_e.shape
    flat = z_e.reshape(-1, Dd)
    idx2, dmin2 = _dist_argmin(flat, embedding)
    indices = idx2.reshape(-1)
    z_q, hist = _sc_gather_hist(embedding, indices)
    loss, perp = _scalars(dmin2, hist)
    return (z_q.reshape(B, M, Dd), loss[0, 0], indices, perp[0, 0])


# speedup vs baseline: 1.0973x; 1.0973x over previous
import functools
import math

import jax
import jax.numpy as jnp
from jax import lax
from jax.experimental import pallas as pl
from jax.experimental.pallas import tpu as pltpu
from jax.experimental.pallas import tpu_sc as plsc

KK = 8192
DD = 256
BETA_ = 0.25
MT = 1024
KC = 2048

NC = 2
NS = 16
NW = NC * NS
NTOK = 32768
B_PER_W = NTOK // NW
GCH = 128
NCH = B_PER_W // GCH


def _dist_argmin_body(x_ref, e_ref, idx_ref, dmin_ref, en_ref, eb_ref,
                      iota_ref):
    @pl.when(pl.program_id(0) == 0)
    def _():
        en_ref[...] = jnp.sum(e_ref[...] ** 2, axis=1)[None, :]
        eb_ref[...] = e_ref[...].astype(jnp.bfloat16)
        iota_ref[...] = lax.broadcasted_iota(
            jnp.int32, (1, KK), 1).astype(jnp.float32)

    x = x_ref[...]
    xn = jnp.sum(x ** 2, axis=1, keepdims=True)
    xb = (-2.0 * x).astype(jnp.bfloat16)
    dot = lax.dot_general(xb, eb_ref[...], (((1,), (1,)), ((), ())),
                          preferred_element_type=jnp.float32)
    s = (xn + dot) + en_ref[...]
    mv = jnp.min(s, axis=1, keepdims=True)
    li = jnp.min(jnp.where(s == mv, iota_ref[...], float(KK)),
                 axis=1, keepdims=True)
    idx_ref[...] = li.astype(jnp.int32)
    dmin_ref[...] = mv


def _dist_argmin(flat, embedding):
    n = flat.shape[0]
    return pl.pallas_call(
        _dist_argmin_body,
        grid=(n // MT,),
        in_specs=[pl.BlockSpec((MT, DD), lambda i: (i, 0)),
                  pl.BlockSpec((KK, DD), lambda i: (0, 0))],
        out_specs=[pl.BlockSpec((MT, 1), lambda i: (i, 0)),
                   pl.BlockSpec((MT, 1), lambda i: (i, 0))],
        out_shape=[jax.ShapeDtypeStruct((n, 1), jnp.int32),
                   jax.ShapeDtypeStruct((n, 1), jnp.float32)],
        scratch_shapes=[pltpu.VMEM((1, KK), jnp.float32),
                        pltpu.VMEM((KK, DD), jnp.bfloat16),
                        pltpu.VMEM((1, KK), jnp.float32)],
    )(flat, embedding)


def _sc_gather_hist_body(e_hbm, idx_hbm, ones_hbm, zeros_hbm,
                         zq_hbm, hist_hbm,
                         idx_v, rows_v, ones_v, hist_sh, gsem):
    cid = lax.axis_index("c")
    sid = lax.axis_index("s")
    wid = sid * NC + cid

    pltpu.sync_copy(idx_hbm.at[pl.ds(wid * NCH, NCH)], idx_v)
    pltpu.sync_copy(ones_hbm, ones_v)

    @pl.when(sid == 0)
    def _():
        pltpu.sync_copy(zeros_hbm, hist_sh)
    plsc.subcore_barrier()

    base = wid * B_PER_W
    for j in range(NCH):
        pltpu.async_copy(e_hbm.at[idx_v.at[j]], rows_v, gsem).wait()
        pltpu.sync_copy(rows_v, zq_hbm.at[pl.ds(base + j * GCH, GCH)])
        pltpu.sync_copy(ones_v, hist_sh.at[idx_v.at[j]], add=True)

    plsc.subcore_barrier()
    @pl.when(sid == 0)
    def _():
        pltpu.sync_copy(hist_sh, hist_hbm.at[cid])


def _sc_gather_hist(embedding, indices):
    idx2 = indices.reshape(NTOK // GCH, GCH)
    ones = jnp.ones((GCH,), jnp.float32)
    zeros = jnp.zeros((KK,), jnp.float32)
    mesh = plsc.VectorSubcoreMesh(core_axis_name="c", subcore_axis_name="s")
    f = pl.kernel(
        _sc_gather_hist_body,
        mesh=mesh,
        out_type=[jax.ShapeDtypeStruct((NTOK, DD), jnp.float32),
                  jax.ShapeDtypeStruct((NC, KK), jnp.float32)],
        scratch_types=[pltpu.VMEM((NCH, GCH), jnp.int32),
                       pltpu.VMEM((GCH, DD), jnp.float32),
                       pltpu.VMEM((GCH,), jnp.float32),
                       pltpu.VMEM_SHARED((KK,), jnp.float32),
                       pltpu.SemaphoreType.DMA],
    )
    return f(embedding, idx2, ones, zeros)


def _scalars_body(d_ref, h_ref, loss_ref, perp_ref):
    loss = BETA_ * (jnp.sum(d_ref[...]) / (NTOK * DD))
    loss_ref[...] = jnp.reshape(loss, (1, 1))
    u = jnp.sum(h_ref[...], axis=0, keepdims=True)
    total = jnp.maximum(jnp.sum(u), 1e-12)
    probs = jnp.clip(u / total, 1e-12, None)
    perp = jnp.exp(-jnp.sum(probs * jnp.log(probs)))
    perp_ref[...] = jnp.reshape(perp, (1, 1))


def _scalars(dmin2, hist):
    return pl.pallas_call(
        _scalars_body,
        out_shape=[jax.ShapeDtypeStruct((1, 1), jnp.float32),
                   jax.ShapeDtypeStruct((1, 1), jnp.float32)],
    )(dmin2.reshape(NTOK // KK, KK), hist)


def kernel(z_e, embedding):
    B, M, Dd = z_e.shape
    flat = z_e.reshape(-1, Dd)
    idx2, dmin2 = _dist_argmin(flat, embedding)
    indices = idx2.reshape(-1)
    z_q, hist = _sc_gather_hist(embedding, indices)
    loss, perp = _scalars(dmin2, hist)
    return (z_q.reshape(B, M, Dd), loss[0, 0], indices, perp[0, 0])
